# baseline (speedup 1.0000x reference)
import functools

import jax
import jax.numpy as jnp
from jax import lax
from jax.experimental import pallas as pl
from jax.experimental.pallas import tpu as pltpu
from jax.experimental.pallas import tpu_sc as plsc

N = 10000
H = 128
BN = 32
NPAD = 10240
NC, NS = 2, 16
NW = NC * NS
G = 6
DUMP = N
RPT = NPAD // NS
RB = 1024


def _sc_mesh():
    return plsc.VectorSubcoreMesh(
        core_axis_name="c", subcore_axis_name="s", num_cores=NC, num_subcores=NS
    )


_SC_PARAMS = pltpu.CompilerParams(
    use_tc_tiling_on_sc=False, needs_layout_passes=False
)


def _edge_geometry(e_rows):
    base = e_rows // NW
    extra = e_rows % NW
    chunks = [G] * (base // G)
    if base % G:
        chunks.append(base % G)
    return base, extra, chunks


def _stage_edges(e_hbm, base, extra, e_rows, srcv, dstv, w):
    pltpu.sync_copy(e_hbm.at[0, pl.ds(w * base, base)], srcv.at[pl.ds(0, base)])
    pltpu.sync_copy(e_hbm.at[1, pl.ds(w * base, base)], dstv.at[pl.ds(0, base)])
    if extra:
        off = e_rows - extra + lax.min(w, extra - 1)
        pltpu.sync_copy(e_hbm.at[0, pl.ds(off, 1)], srcv.at[pl.ds(base, 1)])
        pltpu.sync_copy(e_hbm.at[1, pl.ds(off, 1)], dstv.at[pl.ds(base, 1)])


def _edge_loop(tbl, srcv, dstv, rows, gsem, ssem, acc, chunks):
    starts = [0]
    for g in chunks:
        starts.append(starts[-1] + g)
    nch = len(chunks)

    def fire_gather(k):
        b = k & 1
        return [
            pltpu.async_copy(
                tbl.at[srcv.at[starts[k] + g, 0]],
                rows[b].at[pl.ds(g * 128, 128)],
                gsem[b],
            )
            for g in range(chunks[k])
        ]

    def drain_fire(k):
        b = k & 1
        out = []
        for g, d in enumerate(gd[k]):
            d.wait()
            out.append(
                pltpu.async_copy(
                    rows[b].at[pl.ds(g * 128, 128)],
                    acc.at[dstv.at[starts[k] + g, 0]],
                    ssem[b],
                    add=True,
                )
            )
        return out

    gd = {0: fire_gather(0)}
    sd = {}
    for k in range(nch):
        if k + 1 < nch:
            if k - 1 >= 0:
                for d in sd[k - 1]:
                    d.wait()
            gd[k + 1] = fire_gather(k + 1)
        sd[k] = drain_fire(k)
    for k in range(max(0, nch - 2), nch):
        for d in sd[k]:
            d.wait()


def _extra_edge(tbl, srcv, dstv, buf, sem, acc, base, extra, w):
    if not extra:
        return

    @pl.when(w < extra)
    def _():
        pltpu.async_copy(
            tbl.at[srcv.at[base, 0]], buf.at[pl.ds(0, 128)], sem
        ).wait()
        pltpu.async_copy(
            buf.at[pl.ds(0, 128)], acc.at[dstv.at[base, 0]], sem, add=True
        ).wait()


def _deg_pass(e2, ones_r, z16, e_rows):
    base, extra, chunks = _edge_geometry(e_rows)

    @functools.partial(
        pl.kernel,
        out_type=jax.ShapeDtypeStruct((NC, NPAD, 16), jnp.float32),
        mesh=_sc_mesh(),
        scratch_types=[
            pltpu.VMEM((base + 1, 1, 128), jnp.int32),
            pltpu.VMEM((128, 16), jnp.float32),
            pltpu.VMEM_SHARED((NPAD, 16), jnp.float32),
            pltpu.SemaphoreType.DMA,
        ],
        compiler_params=_SC_PARAMS,
    )
    def deg_kernel(e_hbm, ones_hbm, z_hbm, out_hbm, dstv, ones_v, acc, sem):
        c = lax.axis_index("c")
        s = lax.axis_index("s")
        w = c * NS + s
        r0 = s * RPT

        @pl.when(c == 0)
        def _():
            pltpu.sync_copy(ones_hbm, acc.at[pl.ds(r0, RPT)])

        @pl.when(c != 0)
        def _():
            pltpu.sync_copy(z_hbm, acc.at[pl.ds(r0, RPT)])

        pltpu.sync_copy(ones_hbm.at[pl.ds(0, 128)], ones_v)
        pltpu.sync_copy(e_hbm.at[1, pl.ds(w * base, base)],
                        dstv.at[pl.ds(0, base)])
        if extra:
            off = e_rows - extra + lax.min(w, extra - 1)
            pltpu.sync_copy(e_hbm.at[1, pl.ds(off, 1)], dstv.at[pl.ds(base, 1)])
        plsc.subcore_barrier()
        if extra:
            @pl.when(w < extra)
            def _():
                pltpu.async_copy(ones_v, acc.at[dstv.at[base, 0]], sem,
                                 add=True).wait()
        descs = []
        for r in range(base):
            if r >= 12:
                descs[r - 12].wait()
            descs.append(
                pltpu.async_copy(ones_v, acc.at[dstv.at[r, 0]], sem, add=True)
            )
        for d in descs[-12:]:
            d.wait()
        plsc.subcore_barrier()
        pltpu.sync_copy(
            acc.at[pl.ds(r0, RPT)], out_hbm.at[c, pl.ds(r0, RPT)]
        )

    return deg_kernel(e2, ones_r, z16)


def _conv_pass(tbl, e2, z32, e_rows):
    base, extra, chunks = _edge_geometry(e_rows)
    chunk_max = max(chunks) * 128

    @functools.partial(
        pl.kernel,
        out_type=jax.ShapeDtypeStruct((NC, NPAD, BN), jnp.float32),
        mesh=_sc_mesh(),
        scratch_types=[
            pltpu.VMEM((base + 1, 1, 128), jnp.int32),
            pltpu.VMEM((base + 1, 1, 128), jnp.int32),
            pltpu.VMEM((chunk_max, BN), jnp.float32),
            pltpu.VMEM((chunk_max, BN), jnp.float32),
            pltpu.SemaphoreType.DMA,
            pltpu.SemaphoreType.DMA,
            pltpu.SemaphoreType.DMA,
            pltpu.SemaphoreType.DMA,
            pltpu.VMEM_SHARED((NPAD, BN), jnp.float32),
            pltpu.VMEM_SHARED((NPAD, BN), jnp.float32),
        ],
        compiler_params=_SC_PARAMS,
    )
    def conv_kernel(tbl_hbm, e_hbm, z_hbm, out_hbm,
                    srcv, dstv, rows0, rows1,
                    gsem0, gsem1, ssem0, ssem1, acc, tbl_sp):
        c = lax.axis_index("c")
        s = lax.axis_index("s")
        w = c * NS + s
        r0 = s * RPT
        _stage_edges(e_hbm, base, extra, e_rows, srcv, dstv, w)
        pltpu.sync_copy(tbl_hbm.at[pl.ds(r0, RPT)], tbl_sp.at[pl.ds(r0, RPT)])

        @pl.when(c == 0)
        def _():
            pltpu.sync_copy(tbl_hbm.at[pl.ds(r0, RPT)], acc.at[pl.ds(r0, RPT)])

        @pl.when(c != 0)
        def _():
            pltpu.sync_copy(z_hbm, acc.at[pl.ds(r0, RPT)])

        plsc.subcore_barrier()
        _extra_edge(tbl_sp, srcv, dstv, rows1, gsem1, acc, base, extra, w)
        _edge_loop(tbl_sp, srcv, dstv, (rows0, rows1),
                   (gsem0, gsem1), (ssem0, ssem1), acc, chunks)
        plsc.subcore_barrier()
        pltpu.sync_copy(
            acc.at[pl.ds(r0, RPT)], out_hbm.at[c, pl.ds(r0, RPT)]
        )

    return conv_kernel(tbl, e2, z32)


def _tc_down(x, w_down):
    def body(x_ref, wd_ref, h0_ref):
        h0_ref[...] = lax.dot_general(
            x_ref[...], wd_ref[...], (((1,), (1,)), ((), ())),
            preferred_element_type=jnp.float32,
        )

    return pl.pallas_call(
        body,
        grid=(NPAD // RB,),
        in_specs=[
            pl.BlockSpec((RB, H), lambda i: (i, 0)),
            pl.BlockSpec((BN, H), lambda i: (0, 0)),
        ],
        out_specs=pl.BlockSpec((RB, BN), lambda i: (i, 0)),
        out_shape=jax.ShapeDtypeStruct((NPAD, BN), jnp.float32),
    )(x, w_down)


def _tc_up(m2, w_up, b_up_row, x):
    def body(m2_ref, wu_ref, b_ref, x_ref, out_ref):
        y = lax.dot_general(
            m2_ref[...], wu_ref[...], (((1,), (1,)), ((), ())),
            preferred_element_type=jnp.float32,
        )
        out_ref[...] = y + b_ref[...] + x_ref[...]

    return pl.pallas_call(
        body,
        grid=(NPAD // RB,),
        in_specs=[
            pl.BlockSpec((RB, BN), lambda i: (i, 0)),
            pl.BlockSpec((H, BN), lambda i: (0, 0)),
            pl.BlockSpec((1, H), lambda i: (0, 0)),
            pl.BlockSpec((RB, H), lambda i: (i, 0)),
        ],
        out_specs=pl.BlockSpec((RB, H), lambda i: (i, 0)),
        out_shape=jax.ShapeDtypeStruct((N, H), jnp.float32),
    )(m2, w_up, b_up_row, x)


def kernel(x, edge_index, W_down, b_down, W_up, b_up):
    f32 = jnp.float32
    e = edge_index.shape[1]
    if e % 128:
        npad_e = 128 - e % 128
        pad = jnp.stack([
            jnp.zeros((npad_e,), jnp.int32),
            jnp.full((npad_e,), DUMP, jnp.int32),
        ])
        edge_index = jnp.concatenate([edge_index, pad], axis=1)
        e += npad_e
    e_rows = e // 128
    e2 = edge_index.reshape(2, e_rows, 1, 128)
    z16 = jnp.zeros((RPT, 16), f32)
    z32 = jnp.zeros((RPT, BN), f32)
    ones_r = jnp.ones((RPT, 16), f32)

    degp = _deg_pass(e2, ones_r, z16, e_rows)
    h0 = _tc_down(x, W_down)
    dis = lax.rsqrt(jnp.maximum(degp[0, :, :1] + degp[1, :, :1], 1.0))
    m1p = _conv_pass(h0 * dis, e2, z32, e_rows)
    hs = jnp.maximum((m1p[0] + m1p[1]) * dis + b_down[None, :], 0.0) * dis
    m2p = _conv_pass(hs, e2, z32, e_rows)
    m2 = (m2p[0] + m2p[1]) * dis
    return _tc_up(m2, W_up, b_up.reshape(1, H), x)


# speedup vs baseline: 1.0949x; 1.0949x over previous
import functools

import jax
import jax.numpy as jnp
from jax import lax
from jax.experimental import pallas as pl
from jax.experimental.pallas import tpu as pltpu
from jax.experimental.pallas import tpu_sc as plsc

N = 10000
H = 128
BN = 32
NPAD = 10240
NC, NS = 2, 16
NW = NC * NS
G = 6
DUMP = N
RPT = NPAD // NS
RB = 1024


def _sc_mesh():
    return plsc.VectorSubcoreMesh(
        core_axis_name="c", subcore_axis_name="s", num_cores=NC, num_subcores=NS
    )


_SC_PARAMS = pltpu.CompilerParams(
    use_tc_tiling_on_sc=False, needs_layout_passes=False
)


def _edge_geometry(e_rows):
    base = e_rows // NW
    extra = e_rows % NW
    chunks = [G] * (base // G)
    if base % G:
        chunks.append(base % G)
    return base, extra, chunks


def _stage_edges(e_hbm, base, extra, e_rows, srcv, dstv, w):
    pltpu.sync_copy(e_hbm.at[0, pl.ds(w * base, base)], srcv.at[pl.ds(0, base)])
    pltpu.sync_copy(e_hbm.at[1, pl.ds(w * base, base)], dstv.at[pl.ds(0, base)])
    if extra:
        off = e_rows - extra + lax.min(w, extra - 1)
        pltpu.sync_copy(e_hbm.at[0, pl.ds(off, 1)], srcv.at[pl.ds(base, 1)])
        pltpu.sync_copy(e_hbm.at[1, pl.ds(off, 1)], dstv.at[pl.ds(base, 1)])


def _edge_loop(tbl, srcv, dstv, rows, gsem, ssem, acc, chunks):
    starts = [0]
    for g in chunks:
        starts.append(starts[-1] + g)
    nch = len(chunks)

    def fire_gather(k):
        b = k & 1
        return [
            pltpu.async_copy(
                tbl.at[srcv.at[starts[k] + g, 0]],
                rows[b].at[pl.ds(g * 128, 128)],
                gsem[b],
            )
            for g in range(chunks[k])
        ]

    def drain_fire(k):
        b = k & 1
        out = []
        for g, d in enumerate(gd[k]):
            d.wait()
            out.append(
                pltpu.async_copy(
                    rows[b].at[pl.ds(g * 128, 128)],
                    acc.at[dstv.at[starts[k] + g, 0]],
                    ssem[b],
                    add=True,
                )
            )
        return out

    gd = {0: fire_gather(0)}
    sd = {}
    for k in range(nch):
        if k + 1 < nch:
            if k - 1 >= 0:
                for d in sd[k - 1]:
                    d.wait()
            gd[k + 1] = fire_gather(k + 1)
        sd[k] = drain_fire(k)
    for k in range(max(0, nch - 2), nch):
        for d in sd[k]:
            d.wait()


def _extra_edge(tbl, srcv, dstv, buf, sem, acc, base, extra, w):
    if not extra:
        return

    @pl.when(w < extra)
    def _():
        pltpu.async_copy(
            tbl.at[srcv.at[base, 0]], buf.at[pl.ds(0, 128)], sem
        ).wait()
        pltpu.async_copy(
            buf.at[pl.ds(0, 128)], acc.at[dstv.at[base, 0]], sem, add=True
        ).wait()


def _deg_pass(e2, ones_r, z16, e_rows):
    base, extra, chunks = _edge_geometry(e_rows)

    @functools.partial(
        pl.kernel,
        out_type=jax.ShapeDtypeStruct((NC, NPAD, 16), jnp.float32),
        mesh=_sc_mesh(),
        scratch_types=[
            pltpu.VMEM((base + 1, 1, 128), jnp.int32),
            pltpu.VMEM((128, 16), jnp.float32),
            pltpu.VMEM_SHARED((NPAD, 16), jnp.float32),
            pltpu.SemaphoreType.DMA,
        ],
        compiler_params=_SC_PARAMS,
    )
    def deg_kernel(e_hbm, ones_hbm, z_hbm, out_hbm, dstv, ones_v, acc, sem):
        c = lax.axis_index("c")
        s = lax.axis_index("s")
        w = c * NS + s
        r0 = s * RPT

        @pl.when(c == 0)
        def _():
            pltpu.sync_copy(ones_hbm, acc.at[pl.ds(r0, RPT)])

        @pl.when(c != 0)
        def _():
            pltpu.sync_copy(z_hbm, acc.at[pl.ds(r0, RPT)])

        pltpu.sync_copy(ones_hbm.at[pl.ds(0, 128)], ones_v)
        pltpu.sync_copy(e_hbm.at[1, pl.ds(w * base, base)],
                        dstv.at[pl.ds(0, base)])
        if extra:
            off = e_rows - extra + lax.min(w, extra - 1)
            pltpu.sync_copy(e_hbm.at[1, pl.ds(off, 1)], dstv.at[pl.ds(base, 1)])
        plsc.subcore_barrier()
        if extra:
            @pl.when(w < extra)
            def _():
                pltpu.async_copy(ones_v, acc.at[dstv.at[base, 0]], sem,
                                 add=True).wait()
        descs = []
        for r in range(base):
            if r >= 12:
                descs[r - 12].wait()
            descs.append(
                pltpu.async_copy(ones_v, acc.at[dstv.at[r, 0]], sem, add=True)
            )
        for d in descs[-12:]:
            d.wait()
        plsc.subcore_barrier()
        pltpu.sync_copy(
            acc.at[pl.ds(r0, RPT)], out_hbm.at[c, pl.ds(r0, RPT)]
        )

    return deg_kernel(e2, ones_r, z16)


def _conv_pass(tbl, e2, z32, e_rows):
    base, extra, chunks = _edge_geometry(e_rows)
    chunk_max = max(chunks) * 128

    @functools.partial(
        pl.kernel,
        out_type=jax.ShapeDtypeStruct((NC, NPAD, BN), jnp.float32),
        mesh=_sc_mesh(),
        scratch_types=[
            pltpu.VMEM((base + 1, 1, 128), jnp.int32),
            pltpu.VMEM((base + 1, 1, 128), jnp.int32),
            pltpu.VMEM((chunk_max, BN), jnp.float32),
            pltpu.VMEM((chunk_max, BN), jnp.float32),
            pltpu.SemaphoreType.DMA,
            pltpu.SemaphoreType.DMA,
            pltpu.SemaphoreType.DMA,
            pltpu.SemaphoreType.DMA,
            pltpu.VMEM_SHARED((NPAD, BN), jnp.float32),
        ],
        compiler_params=_SC_PARAMS,
    )
    def conv_kernel(tbl_hbm, e_hbm, z_hbm, out_hbm,
                    srcv, dstv, rows0, rows1,
                    gsem0, gsem1, ssem0, ssem1, acc):
        c = lax.axis_index("c")
        s = lax.axis_index("s")
        w = c * NS + s
        r0 = s * RPT
        _stage_edges(e_hbm, base, extra, e_rows, srcv, dstv, w)

        @pl.when(c == 0)
        def _():
            pltpu.sync_copy(tbl_hbm.at[pl.ds(r0, RPT)], acc.at[pl.ds(r0, RPT)])

        @pl.when(c != 0)
        def _():
            pltpu.sync_copy(z_hbm, acc.at[pl.ds(r0, RPT)])

        plsc.subcore_barrier()
        _extra_edge(tbl_hbm, srcv, dstv, rows1, gsem1, acc, base, extra, w)
        _edge_loop(tbl_hbm, srcv, dstv, (rows0, rows1),
                   (gsem0, gsem1), (ssem0, ssem1), acc, chunks)
        plsc.subcore_barrier()
        pltpu.sync_copy(
            acc.at[pl.ds(r0, RPT)], out_hbm.at[c, pl.ds(r0, RPT)]
        )

    return conv_kernel(tbl, e2, z32)


def _tc_down(x, w_down):
    def body(x_ref, wd_ref, h0_ref):
        h0_ref[...] = lax.dot_general(
            x_ref[...], wd_ref[...], (((1,), (1,)), ((), ())),
            preferred_element_type=jnp.float32,
        )

    return pl.pallas_call(
        body,
        grid=(NPAD // RB,),
        in_specs=[
            pl.BlockSpec((RB, H), lambda i: (i, 0)),
            pl.BlockSpec((BN, H), lambda i: (0, 0)),
        ],
        out_specs=pl.BlockSpec((RB, BN), lambda i: (i, 0)),
        out_shape=jax.ShapeDtypeStruct((NPAD, BN), jnp.float32),
    )(x, w_down)


def _tc_up(m2, w_up, b_up_row, x):
    def body(m2_ref, wu_ref, b_ref, x_ref, out_ref):
        y = lax.dot_general(
            m2_ref[...], wu_ref[...], (((1,), (1,)), ((), ())),
            preferred_element_type=jnp.float32,
        )
        out_ref[...] = y + b_ref[...] + x_ref[...]

    return pl.pallas_call(
        body,
        grid=(NPAD // RB,),
        in_specs=[
            pl.BlockSpec((RB, BN), lambda i: (i, 0)),
            pl.BlockSpec((H, BN), lambda i: (0, 0)),
            pl.BlockSpec((1, H), lambda i: (0, 0)),
            pl.BlockSpec((RB, H), lambda i: (i, 0)),
        ],
        out_specs=pl.BlockSpec((RB, H), lambda i: (i, 0)),
        out_shape=jax.ShapeDtypeStruct((N, H), jnp.float32),
    )(m2, w_up, b_up_row, x)


def kernel(x, edge_index, W_down, b_down, W_up, b_up):
    f32 = jnp.float32
    e = edge_index.shape[1]
    if e % 128:
        npad_e = 128 - e % 128
        pad = jnp.stack([
            jnp.zeros((npad_e,), jnp.int32),
            jnp.full((npad_e,), DUMP, jnp.int32),
        ])
        edge_index = jnp.concatenate([edge_index, pad], axis=1)
        e += npad_e
    e_rows = e // 128
    e2 = edge_index.reshape(2, e_rows, 1, 128)
    z16 = jnp.zeros((RPT, 16), f32)
    z32 = jnp.zeros((RPT, BN), f32)
    ones_r = jnp.ones((RPT, 16), f32)

    degp = _deg_pass(e2, ones_r, z16, e_rows)
    h0 = _tc_down(x, W_down)
    dis = lax.rsqrt(jnp.maximum(degp[0, :, :1] + degp[1, :, :1], 1.0))
    m1p = _conv_pass(h0 * dis, e2, z32, e_rows)
    hs = jnp.maximum((m1p[0] + m1p[1]) * dis + b_down[None, :], 0.0) * dis
    m2p = _conv_pass(hs, e2, z32, e_rows)
    m2 = (m2p[0] + m2p[1]) * dis
    return _tc_up(m2, W_up, b_up.reshape(1, H), x)
